# baseline (speedup 1.0000x reference)
import functools

import jax
import jax.numpy as jnp
from jax import lax
from jax.experimental import pallas as pl
from jax.experimental.pallas import tpu as pltpu
from jax.experimental.pallas import tpu_sc as plsc

B = 16384
EMB = 64
PAIR = 2 * EMB
NC = 2
NS = 16
NW = NC * NS
BPW = B // NW
CHUNK = 128
NCHUNK = BPW // CHUNK
LANES = 16


def _sc_gather(uid2d, iid2d, ut_pairs, it_pairs):
    mesh = plsc.VectorSubcoreMesh(core_axis_name="c", subcore_axis_name="s")

    @functools.partial(
        pl.kernel,
        out_type=[
            jax.ShapeDtypeStruct((B, PAIR), jnp.float32),
            jax.ShapeDtypeStruct((B, PAIR), jnp.float32),
        ],
        mesh=mesh,
        scratch_types=[
            pltpu.VMEM((NCHUNK, CHUNK), jnp.int32),
            pltpu.VMEM((NCHUNK, CHUNK), jnp.int32),
            pltpu.VMEM((2, CHUNK, PAIR), jnp.float32),
            pltpu.VMEM((NCHUNK, CHUNK), jnp.int32),
            pltpu.VMEM((NCHUNK, CHUNK), jnp.int32),
            pltpu.VMEM((2, CHUNK, PAIR), jnp.float32),
            pltpu.SemaphoreType.DMA,
            pltpu.SemaphoreType.DMA,
        ],
    )
    def k(uid_hbm, iid_hbm, ut_hbm, it_hbm, uout, iout,
          uidx, uq, ubuf, iidx, iq, ibuf, usem, isem):
        wid = lax.axis_index("s") * NC + lax.axis_index("c")
        base = wid * BPW
        rowbase = wid * NCHUNK
        pltpu.sync_copy(uid_hbm.at[pl.ds(rowbase, NCHUNK)], uidx)
        pltpu.sync_copy(iid_hbm.at[pl.ds(rowbase, NCHUNK)], iidx)
        for j in range(NCHUNK):
            for g in range(CHUNK // LANES):
                s = pl.ds(g * LANES, LANES)
                uq[j, s] = lax.shift_right_logical(uidx[j, s], 1)
                iq[j, s] = lax.shift_right_logical(iidx[j, s], 1)
        copies = {}

        def start(j):
            copies[("u", j)] = pltpu.async_copy(
                ut_hbm.at[uq.at[j]], ubuf.at[j % 2], usem)
            copies[("i", j)] = pltpu.async_copy(
                it_hbm.at[iq.at[j]], ibuf.at[j % 2], isem)

        start(0)
        for j in range(NCHUNK):
            if j + 1 < NCHUNK:
                start(j + 1)
            copies[("u", j)].wait()
            pltpu.sync_copy(ubuf.at[j % 2],
                            uout.at[pl.ds(base + j * CHUNK, CHUNK)])
            copies[("i", j)].wait()
            pltpu.sync_copy(ibuf.at[j % 2],
                            iout.at[pl.ds(base + j * CHUNK, CHUNK)])

    return k(uid2d, iid2d, ut_pairs, it_pairs)


def _mlp_body(u_ref, i_ref, uid_ref, iid_ref, w1_ref, b1_ref, w2_ref, b2_ref,
              w3_ref, b3_ref, w4_ref, b4_ref, o_ref):
    dn = (((1,), (1,)), ((), ()))
    uodd = (uid_ref[...] & 1) == 1
    iodd = (iid_ref[...] & 1) == 1
    u = jnp.where(uodd, u_ref[:, EMB:], u_ref[:, :EMB])
    i = jnp.where(iodd, i_ref[:, EMB:], i_ref[:, :EMB])
    x = jnp.concatenate([u, i], axis=1)
    h = lax.dot_general(x, w1_ref[...], dn, preferred_element_type=jnp.float32)
    h = jnp.maximum(h + b1_ref[...], 0.0)
    h = lax.dot_general(h, w2_ref[...], dn, preferred_element_type=jnp.float32)
    h = jnp.maximum(h + b2_ref[...], 0.0)
    h = lax.dot_general(h, w3_ref[...], dn, preferred_element_type=jnp.float32)
    h = jnp.maximum(h + b3_ref[...], 0.0)
    o = jnp.sum(h * w4_ref[...], axis=1, keepdims=True) + b4_ref[...]
    o_ref[...] = o


def _tc_mlp(u_emb, i_emb, uid, iid, W1, b1, W2, b2, W3, b3, W4, b4, blk=2048):
    grid = (B // blk,)
    full = lambda b: (0, 0)
    return pl.pallas_call(
        _mlp_body,
        grid=grid,
        in_specs=[
            pl.BlockSpec((blk, PAIR), lambda b: (b, 0)),
            pl.BlockSpec((blk, PAIR), lambda b: (b, 0)),
            pl.BlockSpec((blk, 1), lambda b: (b, 0)),
            pl.BlockSpec((blk, 1), lambda b: (b, 0)),
            pl.BlockSpec(W1.shape, full),
            pl.BlockSpec((1, 256), full),
            pl.BlockSpec(W2.shape, full),
            pl.BlockSpec((1, 128), full),
            pl.BlockSpec(W3.shape, full),
            pl.BlockSpec((1, 64), full),
            pl.BlockSpec(W4.shape, full),
            pl.BlockSpec((1, 1), full),
        ],
        out_specs=pl.BlockSpec((blk, 1), lambda b: (b, 0)),
        out_shape=jax.ShapeDtypeStruct((B, 1), jnp.float32),
    )(u_emb, i_emb, uid, iid, W1, b1.reshape(1, 256), W2, b2.reshape(1, 128),
      W3, b3.reshape(1, 64), W4, b4.reshape(1, 1))


def kernel(user_ids, item_ids, user_table, item_table,
           W1, b1, W2, b2, W3, b3, W4, b4):
    uid = user_ids.astype(jnp.int32)
    iid = item_ids.astype(jnp.int32)
    uid2d = uid.reshape(NW * NCHUNK, CHUNK)
    iid2d = iid.reshape(NW * NCHUNK, CHUNK)
    ut_pairs = user_table.reshape(-1, PAIR)
    it_pairs = item_table.reshape(-1, PAIR)
    u_emb, i_emb = _sc_gather(uid2d, iid2d, ut_pairs, it_pairs)
    out = _tc_mlp(u_emb, i_emb, uid.reshape(B, 1), iid.reshape(B, 1),
                  W1, b1, W2, b2, W3, b3, W4, b4)
    return out[:, 0]


# speedup vs baseline: 1.5515x; 1.5515x over previous
import functools

import jax
import jax.numpy as jnp
from jax import lax
from jax.experimental import pallas as pl
from jax.experimental.pallas import tpu as pltpu
from jax.experimental.pallas import tpu_sc as plsc

B = 16384
EMB = 64
NC = 2
NS = 16
NW = NC * NS
BPW = B // NW
LANES = 16
NG = BPW // LANES
BUFROWS = 256
GPB = 4
NB = BUFROWS // (GPB * LANES)


def _sc_gather(uid2d, iid2d, user_table, item_table):
    mesh = plsc.VectorSubcoreMesh(core_axis_name="c", subcore_axis_name="s")

    @functools.partial(
        pl.kernel,
        out_type=[
            jax.ShapeDtypeStruct((B, EMB), jnp.float32),
            jax.ShapeDtypeStruct((B, EMB), jnp.float32),
        ],
        mesh=mesh,
        scratch_types=[
            pltpu.VMEM((NG, LANES), jnp.int32),
            pltpu.VMEM((NG, LANES), jnp.int32),
            pltpu.VMEM((BUFROWS, EMB), jnp.float32),
            pltpu.SemaphoreType.DMA,
        ],
    )
    def k(uid_hbm, iid_hbm, ut_hbm, it_hbm, uout, iout,
          uidx, iidx, buf, sem):
        wid = lax.axis_index("s") * NC + lax.axis_index("c")
        base = wid * BPW
        rowbase = wid * NG
        pltpu.sync_copy(uid_hbm.at[pl.ds(rowbase, NG)], uidx)
        pltpu.sync_copy(iid_hbm.at[pl.ds(rowbase, NG)], iidx)

        for idx, table, out in ((uidx, ut_hbm, uout), (iidx, it_hbm, iout)):
            for h in range(BPW // BUFROWS):
                g0 = h * (BUFROWS // LANES)

                def body(b, _, idx=idx, table=table, g0=g0):
                    copies = []
                    for g in range(GPB):
                        grp = g0 + b * GPB + g
                        ids = idx[grp, pl.ds(0, LANES)]
                        for l in range(LANES):
                            dst = (b * GPB + g) * LANES + l
                            copies.append(pltpu.async_copy(
                                table.at[pl.ds(ids[l], 1)],
                                buf.at[pl.ds(dst, 1)], sem))
                    for c in copies:
                        c.wait()
                    return 0

                lax.fori_loop(0, NB, body, 0)
                pltpu.sync_copy(
                    buf, out.at[pl.ds(base + h * BUFROWS, BUFROWS)])

    return k(uid2d, iid2d, user_table, item_table)


def _mlp_body(u_ref, i_ref, w1_ref, b1_ref, w2_ref, b2_ref,
              w3_ref, b3_ref, w4_ref, b4_ref, o_ref):
    dn = (((1,), (1,)), ((), ()))
    x = jnp.concatenate([u_ref[...], i_ref[...]], axis=1)
    h = lax.dot_general(x, w1_ref[...], dn, preferred_element_type=jnp.float32)
    h = jnp.maximum(h + b1_ref[...], 0.0)
    h = lax.dot_general(h, w2_ref[...], dn, preferred_element_type=jnp.float32)
    h = jnp.maximum(h + b2_ref[...], 0.0)
    h = lax.dot_general(h, w3_ref[...], dn, preferred_element_type=jnp.float32)
    h = jnp.maximum(h + b3_ref[...], 0.0)
    o = jnp.sum(h * w4_ref[...], axis=1, keepdims=True) + b4_ref[...]
    o_ref[...] = o


def _tc_mlp(u_emb, i_emb, W1, b1, W2, b2, W3, b3, W4, b4, blk=2048):
    grid = (B // blk,)
    full = lambda b: (0, 0)
    return pl.pallas_call(
        _mlp_body,
        grid=grid,
        in_specs=[
            pl.BlockSpec((blk, EMB), lambda b: (b, 0)),
            pl.BlockSpec((blk, EMB), lambda b: (b, 0)),
            pl.BlockSpec(W1.shape, full),
            pl.BlockSpec((1, 256), full),
            pl.BlockSpec(W2.shape, full),
            pl.BlockSpec((1, 128), full),
            pl.BlockSpec(W3.shape, full),
            pl.BlockSpec((1, 64), full),
            pl.BlockSpec(W4.shape, full),
            pl.BlockSpec((1, 1), full),
        ],
        out_specs=pl.BlockSpec((blk, 1), lambda b: (b, 0)),
        out_shape=jax.ShapeDtypeStruct((B, 1), jnp.float32),
    )(u_emb, i_emb, W1, b1.reshape(1, 256), W2, b2.reshape(1, 128),
      W3, b3.reshape(1, 64), W4, b4.reshape(1, 1))


def kernel(user_ids, item_ids, user_table, item_table,
           W1, b1, W2, b2, W3, b3, W4, b4):
    uid2d = user_ids.astype(jnp.int32).reshape(NW * NG, LANES)
    iid2d = item_ids.astype(jnp.int32).reshape(NW * NG, LANES)
    u_emb, i_emb = _sc_gather(uid2d, iid2d, user_table, item_table)
    out = _tc_mlp(u_emb, i_emb, W1, b1, W2, b2, W3, b3, W4, b4)
    return out[:, 0]
